# baseline (speedup 1.0000x reference)
import jax
import jax.numpy as jnp
from jax import lax
from jax.experimental import pallas as pl
from jax.experimental.pallas import tpu as pltpu
from jax.experimental.pallas import tpu_sc as plsc

N_ASU = 2
GRID = 121
G2 = GRID * GRID
G3 = GRID * G2
ASU_SIZE = 2 * 524288
B = 1048576

NC, NS, L = 2, 16, 16
NW = NC * NS
BPW = B // NW
CH = 2048
NCH = BPW // CH
SEEN_SL = ASU_SIZE // NS


def _sc_body(aid_hbm, hkl3_hbm, miller_hbm, seen_hbm,
             out_hbm, seen0_hbm, seen1_hbm,
             asu_v, hkl_v, idx_v, out_v, ones_v, seen_sp, sem):
    c = lax.axis_index("c")
    s = lax.axis_index("s")
    wid = c * NS + s

    @pl.loop(0, CH // L)
    def _ones(i):
        ones_v[pl.ds(i * L, L)] = jnp.ones((L,), jnp.int32)

    pltpu.sync_copy(seen_hbm.at[pl.ds(s * SEEN_SL, SEEN_SL)],
                    seen_sp.at[pl.ds(s * SEEN_SL, SEEN_SL)])
    plsc.subcore_barrier()

    lane3 = lax.iota(jnp.int32, L) * 3

    @pl.loop(0, NCH)
    def _chunk(t):
        base = wid * BPW + t * CH
        stages = [
            pltpu.async_copy(aid_hbm.at[pl.ds(base, CH)], asu_v, sem),
            pltpu.async_copy(hkl3_hbm.at[pl.ds(base * 3, CH * 3)], hkl_v, sem),
        ]
        for st in stages:
            st.wait()

        @pl.loop(0, CH // L)
        def _compute(i):
            off = i * (3 * L) + lane3
            hv = plsc.load_gather(hkl_v, [off])
            kv = plsc.load_gather(hkl_v, [off + 1])
            lv = plsc.load_gather(hkl_v, [off + 2])
            sl = pl.ds(i * L, L)
            idx_v[sl] = asu_v[sl] * G3 + hv * G2 + kv * GRID + lv

        pltpu.async_copy(miller_hbm.at[idx_v], out_v, sem).wait()
        pltpu.sync_copy(ones_v, seen_sp.at[out_v], add=True)
        pltpu.sync_copy(out_v, out_hbm.at[pl.ds(base, CH)])

    plsc.subcore_barrier()
    sl = pl.ds(s * SEEN_SL, SEEN_SL)

    @pl.when(c == 0)
    def _():
        pltpu.sync_copy(seen_sp.at[sl], seen0_hbm.at[sl])

    @pl.when(c == 1)
    def _():
        pltpu.sync_copy(seen_sp.at[sl], seen1_hbm.at[sl])


def _sc_gather_scatter(aid, hkl3, miller, seen_i32):
    mesh = plsc.VectorSubcoreMesh(core_axis_name="c", subcore_axis_name="s")
    f = pl.kernel(
        _sc_body,
        out_type=(jax.ShapeDtypeStruct((B,), jnp.int32),
                  jax.ShapeDtypeStruct((ASU_SIZE,), jnp.int32),
                  jax.ShapeDtypeStruct((ASU_SIZE,), jnp.int32)),
        mesh=mesh,
        compiler_params=pltpu.CompilerParams(needs_layout_passes=False),
        scratch_types=[
            pltpu.VMEM((CH,), jnp.int32),
            pltpu.VMEM((3 * CH,), jnp.int32),
            pltpu.VMEM((CH,), jnp.int32),
            pltpu.VMEM((CH,), jnp.int32),
            pltpu.VMEM((CH,), jnp.int32),
            pltpu.VMEM_SHARED((ASU_SIZE,), jnp.int32),
            pltpu.SemaphoreType.DMA,
        ],
    )
    return f(aid, hkl3, miller, seen_i32)


def _combine_body(s0_ref, s1_ref, o_ref):
    o_ref[...] = (s0_ref[...] | s1_ref[...]) != 0


def _combine(seen0, seen1):
    nrows = ASU_SIZE // 128
    blk = 1024
    spec = pl.BlockSpec((blk, 128), lambda i: (i, 0))
    return pl.pallas_call(
        _combine_body,
        grid=(nrows // blk,),
        in_specs=[spec, spec],
        out_specs=spec,
        out_shape=jax.ShapeDtypeStruct((nrows, 128), jnp.bool_),
    )(seen0.reshape(nrows, 128), seen1.reshape(nrows, 128))


def kernel(asu_id, hkl, miller_id, dHKL, seen):
    del dHKL
    aid = asu_id.reshape(B)
    hkl3 = hkl.reshape(3 * B)
    miller = miller_id.reshape(N_ASU * G3)
    out, seen0, seen1 = _sc_gather_scatter(
        aid, hkl3, miller, seen.astype(jnp.int32))
    seen_new = _combine(seen0, seen1).reshape(ASU_SIZE)
    return out, seen_new


# speedup vs baseline: 9.7753x; 9.7753x over previous
import jax
import jax.numpy as jnp
from jax import lax
from jax.experimental import pallas as pl
from jax.experimental.pallas import tpu as pltpu
from jax.experimental.pallas import tpu_sc as plsc

N_ASU = 2
GRID = 121
G2 = GRID * GRID
G3 = GRID * G2
ASU_SIZE = 2 * 524288
B = 1048576

NC, NS, L = 2, 16, 16
NW = NC * NS
BPW = B // NW
CH = 2048
NCH = BPW // CH
SEEN_SL = ASU_SIZE // NS


def _sc_body(aid_hbm, h_hbm, k_hbm, l_hbm, miller_hbm, seen_hbm,
             out_hbm, seen0_hbm, seen1_hbm,
             a0, h0, k0, l0, i0, o0, a1, h1, k1, l1, i1, o1,
             ones_v, seen_sp, sem_stage, sem_gather, sem_scatter, sem_wout):
    c = lax.axis_index("c")
    s = lax.axis_index("s")
    wid = c * NS + s
    bufs = ((a0, h0, k0, l0, i0, o0), (a1, h1, k1, l1, i1, o1))

    @pl.loop(0, CH // L)
    def _ones(i):
        ones_v[pl.ds(i * L, L)] = jnp.ones((L,), jnp.int32)

    pltpu.sync_copy(seen_hbm.at[pl.ds(s * SEEN_SL, SEEN_SL)],
                    seen_sp.at[pl.ds(s * SEEN_SL, SEEN_SL)])
    plsc.subcore_barrier()

    def stage_pairs(t, par):
        base = wid * BPW + t * CH
        sl = pl.ds(base, CH)
        av, hv, kv, lv = bufs[par][:4]
        return ((aid_hbm.at[sl], av), (h_hbm.at[sl], hv),
                (k_hbm.at[sl], kv), (l_hbm.at[sl], lv))

    def fire_stage(t, par):
        for src, dst in stage_pairs(t, par):
            pltpu.async_copy(src, dst, sem_stage)

    fire_stage(0, 0)

    @pl.loop(0, NCH // 2)
    def _chunks(tt):
        for par in range(2):
            t = tt * 2 + par
            base = wid * BPW + t * CH
            av, hv, kv, lv, iv, ov = bufs[par]

            for src, dst in stage_pairs(t, par):
                pltpu.make_async_copy(src, dst, sem_stage).wait()

            @pl.when(t + 1 < NCH)
            def _():
                fire_stage(t + 1, 1 - par)

            @pl.loop(0, CH // L)
            def _compute(i):
                sl = pl.ds(i * L, L)
                iv[sl] = av[sl] * G3 + hv[sl] * G2 + kv[sl] * GRID + lv[sl]

            @pl.when(t >= 2)
            def _():
                pltpu.make_async_copy(ones_v, seen_sp.at[ov],
                                      sem_scatter).wait()
                pltpu.make_async_copy(ov, out_hbm.at[pl.ds(base, CH)],
                                      sem_wout).wait()

            pltpu.async_copy(miller_hbm.at[iv], ov, sem_gather).wait()

            pltpu.async_copy(ones_v, seen_sp.at[ov], sem_scatter, add=True)
            pltpu.async_copy(ov, out_hbm.at[pl.ds(base, CH)], sem_wout)

    for par in range(2):
        ov = bufs[par][5]
        pltpu.make_async_copy(ones_v, seen_sp.at[ov], sem_scatter).wait()
        pltpu.make_async_copy(ov, out_hbm.at[pl.ds(wid * BPW, CH)],
                              sem_wout).wait()

    plsc.subcore_barrier()
    sl = pl.ds(s * SEEN_SL, SEEN_SL)

    @pl.when(c == 0)
    def _():
        pltpu.sync_copy(seen_sp.at[sl], seen0_hbm.at[sl])

    @pl.when(c == 1)
    def _():
        pltpu.sync_copy(seen_sp.at[sl], seen1_hbm.at[sl])


def _sc_gather_scatter(aid, h, k, l, miller, seen_i32):
    mesh = plsc.VectorSubcoreMesh(core_axis_name="c", subcore_axis_name="s")
    f = pl.kernel(
        _sc_body,
        out_type=(jax.ShapeDtypeStruct((B,), jnp.int32),
                  jax.ShapeDtypeStruct((ASU_SIZE,), jnp.int32),
                  jax.ShapeDtypeStruct((ASU_SIZE,), jnp.int32)),
        mesh=mesh,
        scratch_types=(
            [pltpu.VMEM((CH,), jnp.int32) for _ in range(12)]
            + [pltpu.VMEM((CH,), jnp.int32),
               pltpu.VMEM_SHARED((ASU_SIZE,), jnp.int32),
               pltpu.SemaphoreType.DMA,
               pltpu.SemaphoreType.DMA,
               pltpu.SemaphoreType.DMA,
               pltpu.SemaphoreType.DMA]),
    )
    return f(aid, h, k, l, miller, seen_i32)


def _combine_body(s0_ref, s1_ref, o_ref):
    o_ref[...] = (s0_ref[...] | s1_ref[...]) != 0


def _combine(seen0, seen1):
    nrows = ASU_SIZE // 128
    blk = 1024
    spec = pl.BlockSpec((blk, 128), lambda i: (i, 0))
    return pl.pallas_call(
        _combine_body,
        grid=(nrows // blk,),
        in_specs=[spec, spec],
        out_specs=spec,
        out_shape=jax.ShapeDtypeStruct((nrows, 128), jnp.bool_),
    )(seen0.reshape(nrows, 128), seen1.reshape(nrows, 128))


def kernel(asu_id, hkl, miller_id, dHKL, seen):
    del dHKL
    aid = asu_id.reshape(B)
    hklt = jnp.transpose(hkl)
    miller = miller_id.reshape(N_ASU * G3)
    out, seen0, seen1 = _sc_gather_scatter(
        aid, hklt[0], hklt[1], hklt[2], miller, seen.astype(jnp.int32))
    seen_new = _combine(seen0, seen1).reshape(ASU_SIZE)
    return out, seen_new


# speedup vs baseline: 10.1085x; 1.0341x over previous
import jax
import jax.numpy as jnp
from jax import lax
from jax.experimental import pallas as pl
from jax.experimental.pallas import tpu as pltpu
from jax.experimental.pallas import tpu_sc as plsc

N_ASU = 2
GRID = 121
G2 = GRID * GRID
G3 = GRID * G2
ASU_SIZE = 2 * 524288
B = 1048576

NC, NS, L = 2, 16, 16
NW = NC * NS
BPW = B // NW
CH = 2048
NCH = BPW // CH
SEEN_SL = ASU_SIZE // NS


def _sc_body(aid_hbm, h_hbm, k_hbm, l_hbm, miller_hbm, seen_hbm,
             out_hbm, seen0_hbm, seen1_hbm,
             a0, h0, k0, l0, i0, o0, a1, h1, k1, l1, i1, o1,
             ones_v, seen_sp, sem_stage, sem_gather, sem_scatter, sem_wout):
    c = lax.axis_index("c")
    s = lax.axis_index("s")
    wid = c * NS + s
    bufs = ((a0, h0, k0, l0, i0, o0), (a1, h1, k1, l1, i1, o1))

    @pl.loop(0, CH // L)
    def _ones(i):
        ones_v[pl.ds(i * L, L)] = jnp.ones((L,), jnp.int32)

    pltpu.sync_copy(seen_hbm.at[pl.ds(s * SEEN_SL, SEEN_SL)],
                    seen_sp.at[pl.ds(s * SEEN_SL, SEEN_SL)])
    plsc.subcore_barrier()

    def stage_pairs(t, par):
        base = wid * BPW + t * CH
        sl = pl.ds(base, CH)
        av, hv, kv, lv = bufs[par][:4]
        return ((aid_hbm.at[sl], av), (h_hbm.at[sl], hv),
                (k_hbm.at[sl], kv), (l_hbm.at[sl], lv))

    def fire_stage(t, par):
        for src, dst in stage_pairs(t, par):
            pltpu.async_copy(src, dst, sem_stage)

    fire_stage(0, 0)

    @pl.loop(0, NCH // 2)
    def _chunks(tt):
        for par in range(2):
            t = tt * 2 + par
            base = wid * BPW + t * CH
            av, hv, kv, lv, iv, ov = bufs[par]
            ivp, ovp = bufs[1 - par][4], bufs[1 - par][5]

            for src, dst in stage_pairs(t, par):
                pltpu.make_async_copy(src, dst, sem_stage).wait()

            @pl.when(t + 1 < NCH)
            def _():
                fire_stage(t + 1, 1 - par)

            @pl.loop(0, CH // L)
            def _compute(i):
                sl = pl.ds(i * L, L)
                iv[sl] = av[sl] * G3 + hv[sl] * G2 + kv[sl] * GRID + lv[sl]

            @pl.when(t >= 1)
            def _():
                pltpu.make_async_copy(miller_hbm.at[ivp], ovp,
                                      sem_gather).wait()
                pltpu.async_copy(ones_v, seen_sp.at[ovp], sem_scatter,
                                 add=True)
                pltpu.async_copy(ovp, out_hbm.at[pl.ds(base - CH, CH)],
                                 sem_wout)

            @pl.when(t >= 2)
            def _():
                pltpu.make_async_copy(ones_v, seen_sp.at[ov],
                                      sem_scatter).wait()
                pltpu.make_async_copy(ov, out_hbm.at[pl.ds(base, CH)],
                                      sem_wout).wait()

            pltpu.async_copy(miller_hbm.at[iv], ov, sem_gather)

    iv_last, ov_last = bufs[1][4], bufs[1][5]
    base_last = wid * BPW + (NCH - 1) * CH
    pltpu.make_async_copy(miller_hbm.at[iv_last], ov_last, sem_gather).wait()
    pltpu.async_copy(ones_v, seen_sp.at[ov_last], sem_scatter, add=True)
    pltpu.async_copy(ov_last, out_hbm.at[pl.ds(base_last, CH)], sem_wout)
    for par in range(2):
        ov = bufs[par][5]
        pltpu.make_async_copy(ones_v, seen_sp.at[ov], sem_scatter).wait()
        pltpu.make_async_copy(ov, out_hbm.at[pl.ds(wid * BPW, CH)],
                              sem_wout).wait()

    plsc.subcore_barrier()
    sl = pl.ds(s * SEEN_SL, SEEN_SL)

    @pl.when(c == 0)
    def _():
        pltpu.sync_copy(seen_sp.at[sl], seen0_hbm.at[sl])

    @pl.when(c == 1)
    def _():
        pltpu.sync_copy(seen_sp.at[sl], seen1_hbm.at[sl])


def _sc_gather_scatter(aid, h, k, l, miller, seen_i32):
    mesh = plsc.VectorSubcoreMesh(core_axis_name="c", subcore_axis_name="s")
    f = pl.kernel(
        _sc_body,
        out_type=(jax.ShapeDtypeStruct((B,), jnp.int32),
                  jax.ShapeDtypeStruct((ASU_SIZE,), jnp.int32),
                  jax.ShapeDtypeStruct((ASU_SIZE,), jnp.int32)),
        mesh=mesh,
        scratch_types=(
            [pltpu.VMEM((CH,), jnp.int32) for _ in range(12)]
            + [pltpu.VMEM((CH,), jnp.int32),
               pltpu.VMEM_SHARED((ASU_SIZE,), jnp.int32),
               pltpu.SemaphoreType.DMA,
               pltpu.SemaphoreType.DMA,
               pltpu.SemaphoreType.DMA,
               pltpu.SemaphoreType.DMA]),
    )
    return f(aid, h, k, l, miller, seen_i32)


def _combine_body(s0_ref, s1_ref, o_ref):
    o_ref[...] = (s0_ref[...] | s1_ref[...]) != 0


def _combine(seen0, seen1):
    nrows = ASU_SIZE // 128
    blk = 1024
    spec = pl.BlockSpec((blk, 128), lambda i: (i, 0))
    return pl.pallas_call(
        _combine_body,
        grid=(nrows // blk,),
        in_specs=[spec, spec],
        out_specs=spec,
        out_shape=jax.ShapeDtypeStruct((nrows, 128), jnp.bool_),
    )(seen0.reshape(nrows, 128), seen1.reshape(nrows, 128))


def kernel(asu_id, hkl, miller_id, dHKL, seen):
    del dHKL
    aid = asu_id.reshape(B)
    hklt = jnp.transpose(hkl)
    miller = miller_id.reshape(N_ASU * G3)
    out, seen0, seen1 = _sc_gather_scatter(
        aid, hklt[0], hklt[1], hklt[2], miller, seen.astype(jnp.int32))
    seen_new = _combine(seen0, seen1).reshape(ASU_SIZE)
    return out, seen_new


# speedup vs baseline: 10.3181x; 1.0207x over previous
import jax
import jax.numpy as jnp
from jax import lax
from jax.experimental import pallas as pl
from jax.experimental.pallas import tpu as pltpu
from jax.experimental.pallas import tpu_sc as plsc

N_ASU = 2
GRID = 121
G2 = GRID * GRID
G3 = GRID * G2
ASU_SIZE = 2 * 524288
B = 1048576

NC, NS, L = 2, 16, 16
NW = NC * NS
BPW = B // NW
CH = 4096
NCH = BPW // CH
SEEN_SL = ASU_SIZE // NS


def _sc_body(aid_hbm, h_hbm, k_hbm, l_hbm, miller_hbm, seen_hbm,
             out_hbm, seen0_hbm, seen1_hbm,
             a0, h0, k0, l0, i0, o0, a1, h1, k1, l1, i1, o1,
             ones_v, seen_sp, sem_stage, sem_gather, sem_scatter, sem_wout):
    c = lax.axis_index("c")
    s = lax.axis_index("s")
    wid = c * NS + s
    bufs = ((a0, h0, k0, l0, i0, o0), (a1, h1, k1, l1, i1, o1))

    @pl.loop(0, CH // L)
    def _ones(i):
        ones_v[pl.ds(i * L, L)] = jnp.ones((L,), jnp.int32)

    pltpu.sync_copy(seen_hbm.at[pl.ds(s * SEEN_SL, SEEN_SL)],
                    seen_sp.at[pl.ds(s * SEEN_SL, SEEN_SL)])
    plsc.subcore_barrier()

    def stage_pairs(t, par):
        base = wid * BPW + t * CH
        sl = pl.ds(base, CH)
        av, hv, kv, lv = bufs[par][:4]
        return ((aid_hbm.at[sl], av), (h_hbm.at[sl], hv),
                (k_hbm.at[sl], kv), (l_hbm.at[sl], lv))

    def fire_stage(t, par):
        for src, dst in stage_pairs(t, par):
            pltpu.async_copy(src, dst, sem_stage)

    fire_stage(0, 0)

    @pl.loop(0, NCH // 2)
    def _chunks(tt):
        for par in range(2):
            t = tt * 2 + par
            base = wid * BPW + t * CH
            av, hv, kv, lv, iv, ov = bufs[par]
            ivp, ovp = bufs[1 - par][4], bufs[1 - par][5]

            for src, dst in stage_pairs(t, par):
                pltpu.make_async_copy(src, dst, sem_stage).wait()

            @pl.when(t + 1 < NCH)
            def _():
                fire_stage(t + 1, 1 - par)

            @pl.loop(0, CH // L)
            def _compute(i):
                sl = pl.ds(i * L, L)
                iv[sl] = av[sl] * G3 + hv[sl] * G2 + kv[sl] * GRID + lv[sl]

            @pl.when(t >= 1)
            def _():
                pltpu.make_async_copy(miller_hbm.at[ivp], ovp,
                                      sem_gather).wait()
                pltpu.async_copy(ones_v, seen_sp.at[ovp], sem_scatter,
                                 add=True)
                pltpu.async_copy(ovp, out_hbm.at[pl.ds(base - CH, CH)],
                                 sem_wout)

            @pl.when(t >= 2)
            def _():
                pltpu.make_async_copy(ones_v, seen_sp.at[ov],
                                      sem_scatter).wait()
                pltpu.make_async_copy(ov, out_hbm.at[pl.ds(base, CH)],
                                      sem_wout).wait()

            pltpu.async_copy(miller_hbm.at[iv], ov, sem_gather)

    iv_last, ov_last = bufs[1][4], bufs[1][5]
    base_last = wid * BPW + (NCH - 1) * CH
    pltpu.make_async_copy(miller_hbm.at[iv_last], ov_last, sem_gather).wait()
    pltpu.async_copy(ones_v, seen_sp.at[ov_last], sem_scatter, add=True)
    pltpu.async_copy(ov_last, out_hbm.at[pl.ds(base_last, CH)], sem_wout)
    for par in range(2):
        ov = bufs[par][5]
        pltpu.make_async_copy(ones_v, seen_sp.at[ov], sem_scatter).wait()
        pltpu.make_async_copy(ov, out_hbm.at[pl.ds(wid * BPW, CH)],
                              sem_wout).wait()

    plsc.subcore_barrier()
    sl = pl.ds(s * SEEN_SL, SEEN_SL)

    @pl.when(c == 0)
    def _():
        pltpu.sync_copy(seen_sp.at[sl], seen0_hbm.at[sl])

    @pl.when(c == 1)
    def _():
        pltpu.sync_copy(seen_sp.at[sl], seen1_hbm.at[sl])


def _sc_gather_scatter(aid, h, k, l, miller, seen_i32):
    mesh = plsc.VectorSubcoreMesh(core_axis_name="c", subcore_axis_name="s")
    f = pl.kernel(
        _sc_body,
        out_type=(jax.ShapeDtypeStruct((B,), jnp.int32),
                  jax.ShapeDtypeStruct((ASU_SIZE,), jnp.int32),
                  jax.ShapeDtypeStruct((ASU_SIZE,), jnp.int32)),
        mesh=mesh,
        scratch_types=(
            [pltpu.VMEM((CH,), jnp.int32) for _ in range(12)]
            + [pltpu.VMEM((CH,), jnp.int32),
               pltpu.VMEM_SHARED((ASU_SIZE,), jnp.int32),
               pltpu.SemaphoreType.DMA,
               pltpu.SemaphoreType.DMA,
               pltpu.SemaphoreType.DMA,
               pltpu.SemaphoreType.DMA]),
    )
    return f(aid, h, k, l, miller, seen_i32)


def _combine_body(s0_ref, s1_ref, o_ref):
    o_ref[...] = (s0_ref[...] | s1_ref[...]) != 0


def _combine(seen0, seen1):
    nrows = ASU_SIZE // 128
    blk = 1024
    spec = pl.BlockSpec((blk, 128), lambda i: (i, 0))
    return pl.pallas_call(
        _combine_body,
        grid=(nrows // blk,),
        in_specs=[spec, spec],
        out_specs=spec,
        out_shape=jax.ShapeDtypeStruct((nrows, 128), jnp.bool_),
    )(seen0.reshape(nrows, 128), seen1.reshape(nrows, 128))


def kernel(asu_id, hkl, miller_id, dHKL, seen):
    del dHKL
    aid = asu_id.reshape(B)
    hklt = jnp.transpose(hkl)
    miller = miller_id.reshape(N_ASU * G3)
    out, seen0, seen1 = _sc_gather_scatter(
        aid, hklt[0], hklt[1], hklt[2], miller, seen.astype(jnp.int32))
    seen_new = _combine(seen0, seen1).reshape(ASU_SIZE)
    return out, seen_new
